# baseline (speedup 1.0000x reference)
import numpy as np
import jax
import jax.numpy as jnp
from jax.experimental import pallas as pl
from jax.experimental.pallas import tpu as pltpu

_R, _C = 128, 100000
_BR = 8
_NSTEP = _R // _BR

_U = np.uint32
_K1 = _U(42)
_K2 = _U(0 ^ 42 ^ 0x1BD11BDA)

_ROTS = (13, 15, 26, 6, 17, 29, 16, 24, 13, 15, 26, 6, 17, 29, 16, 24,
         13, 15, 26, 6)
_INJ = {
    4: (_K1, _U(_K2 + _U(1))),
    8: (_K2, _U(2)),
    12: (None, _U(_K1 + _U(3))),
    16: (_K1, _U(_K2 + _U(4))),
    20: (_K2, _U(5)),
}


def _rotl(x, d):
    return (x << _U(d)) | (x >> _U(32 - d))


def _decide(logits, step):
    shape = logits.shape
    row = jax.lax.broadcasted_iota(jnp.int32, shape, 0) + step * _BR
    col = jax.lax.broadcasted_iota(jnp.int32, shape, 1)
    c1 = (row * _C + col).astype(jnp.uint32)

    x1 = c1 + _K1
    x0 = x1
    x1 = x0 ^ _rotl(x1, _ROTS[0])
    for rnd, r in enumerate(_ROTS[1:], start=2):
        x0 = x0 + x1
        x1 = x0 ^ _rotl(x1, r)
        if rnd in _INJ:
            a, b = _INJ[rnd]
            if a is not None:
                x0 = x0 + a
            x1 = x1 + b
    bits = x0 ^ x1

    fb = (bits >> _U(9)) | _U(0x3F800000)
    f = jax.lax.bitcast_convert_type(fb, jnp.float32)
    thr = jnp.exp(-jnp.exp(logits))
    return ((2.0 - f) > thr).astype(jnp.float32)


def _body(x_hbm, o_hbm, xbuf, obuf, insem, outsem):
    def in_copy(i, slot):
        return pltpu.make_async_copy(
            x_hbm.at[pl.ds(i * _BR, _BR)], xbuf.at[slot], insem.at[slot])

    def out_copy(i, slot):
        return pltpu.make_async_copy(
            obuf.at[slot], o_hbm.at[pl.ds(i * _BR, _BR)], outsem.at[slot])

    in_copy(0, 0).start()

    def step(i, carry):
        slot = jax.lax.rem(i, 2)
        nxt = 1 - slot

        @pl.when(i + 1 < _NSTEP)
        def _():
            in_copy(i + 1, nxt).start()

        in_copy(i, slot).wait()

        @pl.when(i >= 2)
        def _():
            out_copy(i - 2, slot).wait()

        obuf[slot] = _decide(xbuf[slot], i)
        out_copy(i, slot).start()
        return carry

    jax.lax.fori_loop(0, _NSTEP, step, 0)
    out_copy(_NSTEP - 2, (_NSTEP - 2) % 2).wait()
    out_copy(_NSTEP - 1, (_NSTEP - 1) % 2).wait()


@jax.jit
def kernel(logits):
    return pl.pallas_call(
        _body,
        out_shape=jax.ShapeDtypeStruct((_R, _C), jnp.float32),
        in_specs=[pl.BlockSpec(memory_space=pl.ANY)],
        out_specs=pl.BlockSpec(memory_space=pl.ANY),
        scratch_shapes=[
            pltpu.VMEM((2, _BR, _C), jnp.float32),
            pltpu.VMEM((2, _BR, _C), jnp.float32),
            pltpu.SemaphoreType.DMA((2,)),
            pltpu.SemaphoreType.DMA((2,)),
        ],
    )(logits)
